# baseline (speedup 1.0000x reference)
import jax
import jax.numpy as jnp
from jax.experimental import pallas as pl
from jax.experimental.pallas import tpu as pltpu

BATCH_BLOCK = 128


def _add_kernel(x_ref, pos_ref, out_ref):
    out_ref[...] = x_ref[...] + pos_ref[...][None]


def kernel(x, pos_table):
    batch, maxlen, dim = x.shape
    grid = (batch // BATCH_BLOCK,)
    return pl.pallas_call(
        _add_kernel,
        grid=grid,
        in_specs=[
            pl.BlockSpec((BATCH_BLOCK, maxlen, dim), lambda i: (i, 0, 0)),
            pl.BlockSpec((maxlen, dim), lambda i: (0, 0)),
        ],
        out_specs=pl.BlockSpec((BATCH_BLOCK, maxlen, dim), lambda i: (i, 0, 0)),
        out_shape=jax.ShapeDtypeStruct((batch, maxlen, dim), x.dtype),
        compiler_params=pltpu.CompilerParams(
            dimension_semantics=("parallel",)),
    )(x, pos_table)


# speedup vs baseline: 5.7942x; 5.7942x over previous
import jax
import jax.numpy as jnp
from jax.experimental import pallas as pl
from jax.experimental.pallas import tpu as pltpu

ROW_BLOCK = 256


def _add_kernel(x_ref, pos_ref, out_ref):
    out_ref[...] = x_ref[...] + pos_ref[...]


def kernel(x, pos_table):
    batch, maxlen, dim = x.shape
    rows = maxlen * dim
    xt = jnp.transpose(x, (1, 2, 0)).reshape(rows, batch)
    pos_col = pos_table.reshape(rows, 1)
    grid = (rows // ROW_BLOCK,)
    out = pl.pallas_call(
        _add_kernel,
        grid=grid,
        in_specs=[
            pl.BlockSpec((ROW_BLOCK, batch), lambda i: (i, 0)),
            pl.BlockSpec((ROW_BLOCK, 1), lambda i: (i, 0)),
        ],
        out_specs=pl.BlockSpec((ROW_BLOCK, batch), lambda i: (i, 0)),
        out_shape=jax.ShapeDtypeStruct((rows, batch), x.dtype),
        compiler_params=pltpu.CompilerParams(
            dimension_semantics=("parallel",)),
    )(xt, pos_col)
    return out.reshape(maxlen, dim, batch).transpose(2, 0, 1)


# speedup vs baseline: 5.8503x; 1.0097x over previous
import jax
import jax.numpy as jnp
from jax.experimental import pallas as pl
from jax.experimental.pallas import tpu as pltpu

ROW_BLOCK = 512


def _add_kernel(x_ref, pos_ref, out_ref):
    out_ref[...] = x_ref[...] + pos_ref[...]


def kernel(x, pos_table):
    batch, maxlen, dim = x.shape
    rows = maxlen * dim
    xt = jnp.transpose(x, (1, 2, 0)).reshape(rows, batch)
    pos_col = pos_table.reshape(rows, 1)
    grid = (rows // ROW_BLOCK,)
    out = pl.pallas_call(
        _add_kernel,
        grid=grid,
        in_specs=[
            pl.BlockSpec((ROW_BLOCK, batch), lambda i: (i, 0)),
            pl.BlockSpec((ROW_BLOCK, 1), lambda i: (i, 0)),
        ],
        out_specs=pl.BlockSpec((ROW_BLOCK, batch), lambda i: (i, 0)),
        out_shape=jax.ShapeDtypeStruct((rows, batch), x.dtype),
        compiler_params=pltpu.CompilerParams(
            dimension_semantics=("parallel",)),
    )(xt, pos_col)
    return out.reshape(maxlen, dim, batch).transpose(2, 0, 1)


# speedup vs baseline: 6.3330x; 1.0825x over previous
import jax
import jax.numpy as jnp
from jax.experimental import pallas as pl
from jax.experimental.pallas import tpu as pltpu

T_BLOCK = 8


def _add_kernel(x_ref, pos_ref, out_ref):
    out_ref[...] = x_ref[...] + pos_ref[...][:, :, None]


def kernel(x, pos_table):
    batch, maxlen, dim = x.shape
    xt = jnp.transpose(x, (1, 2, 0))
    grid = (maxlen // T_BLOCK,)
    out = pl.pallas_call(
        _add_kernel,
        grid=grid,
        in_specs=[
            pl.BlockSpec((T_BLOCK, dim, batch), lambda i: (i, 0, 0)),
            pl.BlockSpec((T_BLOCK, dim), lambda i: (i, 0)),
        ],
        out_specs=pl.BlockSpec((T_BLOCK, dim, batch), lambda i: (i, 0, 0)),
        out_shape=jax.ShapeDtypeStruct((maxlen, dim, batch), x.dtype),
        compiler_params=pltpu.CompilerParams(
            dimension_semantics=("arbitrary",)),
    )(xt, pos_table)
    return out.transpose(2, 0, 1)
